# baseline (speedup 1.0000x reference)
import functools

import jax
import jax.numpy as jnp
from jax import lax
from jax.experimental import pallas as pl
from jax.experimental.pallas import tpu as pltpu
from jax.experimental.pallas import tpu_sc as plsc

N = 4096
D = 128
TM = 1024
TN = 1024
NW = 32
RPW = N // NW
NCH = RPW // 16

CONTRASTIVE_MARGIN = 0.5
TRIPLET_MARGIN = 0.2
ALPHA = 0.5
EPS = 1e-6


def _compact_kernel(e1_hbm, e2_hbm, tgt_hbm,
                    e1p_hbm, e2p_hbm, e2n_hbm, counts_hbm,
                    t_v, d1_v, d0_v, r1_v, r2_v, cnt_v,
                    sem1, sem2, sem3, sem4):
    wid = lax.axis_index("s") * 2 + lax.axis_index("c")
    base = wid * RPW

    pltpu.sync_copy(tgt_hbm, t_v)
    pltpu.sync_copy(e1_hbm.at[pl.ds(base, RPW)], r1_v)
    pltpu.sync_copy(e2_hbm.at[pl.ds(base, RPW)], r2_v)

    def _scan(c, carry):
        tot, pref = carry
        s = jnp.sum(t_v[pl.ds(c * 16, 16)])
        return tot + s, pref + jnp.where(c < base // 16, s, 0)

    tot_pos, pref_pos = lax.fori_loop(0, N // 16, _scan, (0, 0))
    tot_neg = N - tot_pos

    idx16 = lax.iota(jnp.int32, 16)

    def _slots(c, ppref):
        t = t_v[pl.ds(base + c * 16, 16)]
        pc_inc = plsc.cumsum(t)
        pcum = ppref + pc_inc - t
        row = base + c * 16 + idx16
        ncum = row - pcum
        d1 = jnp.where(t == 1, pcum, tot_pos + ncum)
        d0 = jnp.where(t == 0, ncum, tot_neg + pcum)
        d1_v[pl.ds(c * 16, 16)] = d1
        d0_v[pl.ds(c * 16, 16)] = d0
        return ppref + jnp.sum(t)

    lax.fori_loop(0, NCH, _slots, pref_pos)

    c1 = pltpu.async_copy(r1_v, e1p_hbm.at[d1_v], sem1)
    c2 = pltpu.async_copy(r2_v, e2p_hbm.at[d1_v], sem2)
    c3 = pltpu.async_copy(r2_v, e2n_hbm.at[d0_v], sem3)

    @pl.when(wid == 0)
    def _counts():
        cnt_v[...] = jnp.where(idx16 == 0, tot_pos,
                               jnp.where(idx16 == 1, tot_neg, 0))
        pltpu.async_copy(cnt_v, counts_hbm, sem4).wait()

    c1.wait()
    c2.wait()
    c3.wait()


def _compact(emb1, emb2, target):
    mesh = plsc.VectorSubcoreMesh(core_axis_name="c", subcore_axis_name="s")
    f32, i32 = jnp.float32, jnp.int32
    fn = functools.partial(
        pl.kernel, mesh=mesh,
        compiler_params=pltpu.CompilerParams(needs_layout_passes=False),
        out_type=(
            jax.ShapeDtypeStruct((N, D), f32),
            jax.ShapeDtypeStruct((N, D), f32),
            jax.ShapeDtypeStruct((N, D), f32),
            jax.ShapeDtypeStruct((16,), i32),
        ),
        scratch_types=[
            pltpu.VMEM((N,), i32),
            pltpu.VMEM((RPW,), i32),
            pltpu.VMEM((RPW,), i32),
            pltpu.VMEM((RPW, D), f32),
            pltpu.VMEM((RPW, D), f32),
            pltpu.VMEM((16,), i32),
            pltpu.SemaphoreType.DMA,
            pltpu.SemaphoreType.DMA,
            pltpu.SemaphoreType.DMA,
            pltpu.SemaphoreType.DMA,
        ],
    )(_compact_kernel)
    return fn(emb1, emb2, target)


def _hybrid_loss_kernel(counts_ref, a_ref, bp_ref, bnfull_ref, out_ref,
                        bbf_ref, bias_ref, acc_ref):
    i = pl.program_id(0)
    nsteps = pl.num_programs(0)
    npos = counts_ref[0]
    nneg = counts_ref[1]

    @pl.when(i == 0)
    def _init():
        acc_ref[0] = 0.0
        acc_ref[1] = 0.0
        bbf_ref[...] = bnfull_ref[...].astype(jnp.bfloat16)
        colid = lax.broadcasted_iota(jnp.int32, (1, N), 1)
        bias_ref[...] = jnp.where(colid >= nneg, 4.0, 0.0)

    a = a_ref[...]
    b = bp_ref[...]
    rowid = lax.broadcasted_iota(jnp.int32, (1, TM), 1) + i * TM
    tf = (rowid < npos).astype(jnp.float32)

    diff = a - b + EPS
    sq = diff * diff
    ones_row = jnp.ones((1, D), jnp.float32)
    psq = jax.lax.dot_general(ones_row, sq, (((1,), (1,)), ((), ())),
                              preferred_element_type=jnp.float32,
                              precision=jax.lax.Precision.HIGHEST)
    pos_dist = jnp.sqrt(psq)

    margin_gap = jnp.maximum(CONTRASTIVE_MARGIN - pos_dist, 0.0)
    loss_sim = tf * psq
    loss_dis = (1.0 - tf) * margin_gap * margin_gap
    hard = (tf == 0.0) & (pos_dist < CONTRASTIVE_MARGIN)
    w = jnp.where(hard, 2.0, 1.0)
    acc_ref[0] += jnp.sum(0.5 * (loss_sim + loss_dis) * w)

    @pl.when(i * TM < npos)
    def _mine():
        abf = a.astype(jnp.bfloat16)
        njc = (nneg + TN - 1) // TN

        def _chunk(j, rmax):
            bt = bbf_ref[pl.ds(j * TN, TN), :]
            dot = jax.lax.dot_general(abf, bt, (((1,), (1,)), ((), ())),
                                      preferred_element_type=jnp.float32)
            biasc = bias_ref[:, pl.ds(j * TN, TN)]
            return jnp.maximum(
                rmax, jnp.max(dot - biasc, axis=1, keepdims=True))

        rmax0 = jnp.full((TM, 1), -4.0, jnp.float32)
        rmax = lax.fori_loop(0, njc, _chunk, rmax0)

        max_dot = rmax.reshape(1, TM)
        min_d2 = 2.0 - 2.0 * max_dot
        neg_dist = jnp.sqrt(jnp.maximum(min_d2, 1e-12))
        tl = jnp.maximum(pos_dist - neg_dist + TRIPLET_MARGIN, 0.0) * tf
        acc_ref[1] += jnp.sum(tl)

    @pl.when(i == nsteps - 1)
    def _finish():
        npf = npos.astype(jnp.float32)
        contr = acc_ref[0] / N
        has_both = (npos > 0) & (nneg > 0)
        trip = jnp.where(has_both, acc_ref[1] / jnp.maximum(npf, 1.0), 0.0)
        out_ref[0] = ALPHA * contr + (1.0 - ALPHA) * trip


def kernel(emb1, emb2, target):
    e1p, e2p, e2n, counts = _compact(emb1, emb2, target)

    out = pl.pallas_call(
        _hybrid_loss_kernel,
        grid=(N // TM,),
        in_specs=[
            pl.BlockSpec(memory_space=pltpu.SMEM),
            pl.BlockSpec((TM, D), lambda i: (i, 0)),
            pl.BlockSpec((TM, D), lambda i: (i, 0)),
            pl.BlockSpec((N, D), lambda i: (0, 0)),
        ],
        out_specs=pl.BlockSpec(memory_space=pltpu.SMEM),
        out_shape=jax.ShapeDtypeStruct((1,), jnp.float32),
        scratch_shapes=[
            pltpu.VMEM((N, D), jnp.bfloat16),
            pltpu.VMEM((1, N), jnp.float32),
            pltpu.SMEM((2,), jnp.float32),
        ],
    )(counts, e1p, e2p, e2n)
    return out[0]


# speedup vs baseline: 3.0067x; 3.0067x over previous
import jax
import jax.numpy as jnp
from jax.experimental import pallas as pl
from jax.experimental.pallas import tpu as pltpu

N = 4096
D = 128
TM = 2048

CONTRASTIVE_MARGIN = 0.5
TRIPLET_MARGIN = 0.2
ALPHA = 0.5
EPS = 1e-6


def _hybrid_loss_kernel(a_ref, bfull_ref, trow_ref, out_ref,
                        bias_ref, acc_ref):
    i = pl.program_id(0)
    nsteps = pl.num_programs(0)

    @pl.when(i == 0)
    def _init():
        acc_ref[0] = 0.0
        acc_ref[1] = 0.0
        acc_ref[2] = 0.0
        bias_ref[...] = 4.0 * trow_ref[...].astype(jnp.float32)

    a = a_ref[...]
    b = bfull_ref[pl.ds(i * TM, TM), :]
    tf = trow_ref[:, pl.ds(i * TM, TM)].astype(jnp.float32)

    diff = a - b + EPS
    sq = diff * diff
    ones_row = jnp.ones((1, D), jnp.float32)
    psq = jax.lax.dot_general(ones_row, sq, (((1,), (1,)), ((), ())),
                              preferred_element_type=jnp.float32)
    pos_dist = jnp.sqrt(psq)

    margin_gap = jnp.maximum(CONTRASTIVE_MARGIN - pos_dist, 0.0)
    loss_sim = tf * psq
    loss_dis = (1.0 - tf) * margin_gap * margin_gap
    hard = (tf == 0.0) & (pos_dist < CONTRASTIVE_MARGIN)
    w = jnp.where(hard, 2.0, 1.0)
    c_sum = jnp.sum(0.5 * (loss_sim + loss_dis) * w)

    dot = jax.lax.dot_general(a, bfull_ref[...], (((1,), (1,)), ((), ())),
                              preferred_element_type=jnp.float32)
    rmax = jnp.max(dot - bias_ref[...], axis=1, keepdims=True)
    max_dot = rmax.reshape(1, TM)
    min_d2 = 2.0 - 2.0 * max_dot
    neg_dist = jnp.sqrt(jnp.maximum(min_d2, 1e-12))
    tl = jnp.maximum(pos_dist - neg_dist + TRIPLET_MARGIN, 0.0) * tf
    t_sum = jnp.sum(tl)
    p_sum = jnp.sum(tf)

    acc_ref[0] += c_sum
    acc_ref[1] += t_sum
    acc_ref[2] += p_sum

    @pl.when(i == nsteps - 1)
    def _finish():
        npos = acc_ref[2]
        contr = acc_ref[0] / N
        has_both = (npos > 0.5) & (npos < N - 0.5)
        trip = jnp.where(has_both, acc_ref[1] / jnp.maximum(npos, 1.0), 0.0)
        out_ref[0] = ALPHA * contr + (1.0 - ALPHA) * trip


def kernel(emb1, emb2, target):
    trow = target.reshape(1, N)

    out = pl.pallas_call(
        _hybrid_loss_kernel,
        grid=(N // TM,),
        in_specs=[
            pl.BlockSpec((TM, D), lambda i: (i, 0)),
            pl.BlockSpec((N, D), lambda i: (0, 0)),
            pl.BlockSpec((1, N), lambda i: (0, 0)),
        ],
        out_specs=pl.BlockSpec(memory_space=pltpu.SMEM),
        out_shape=jax.ShapeDtypeStruct((1,), jnp.float32),
        scratch_shapes=[
            pltpu.VMEM((1, N), jnp.float32),
            pltpu.SMEM((3,), jnp.float32),
        ],
    )(emb1, emb2, trow)
    return out[0]
